# Initial kernel scaffold; baseline (speedup 1.0000x reference)
import jax
import jax.numpy as jnp
from jax.experimental import pallas as pl


def kernel(logits, p, k, a, m, temperatures):
    raise NotImplementedError("write your pallas kernel here")


# speedup vs baseline: 20.4868x; 20.4868x over previous
import functools

import jax
import jax.numpy as jnp
from jax import lax
from jax.experimental import pallas as pl
from jax.experimental.pallas import tpu as pltpu
from jax.experimental.pallas import tpu_sc as plsc

L = 16
NC, NS = 2, 16
NW = NC * NS

NBINS = 256
CAP = 2048
TOPK = 1024
RADIX_BITS = 8
NRAD = 1 << RADIX_BITS
NPASS = 4
ZB = 16384

_I32_MIN = -2147483648
_NEG_INF = float("-inf")


def _vfull(x, dtype=jnp.float32):
  return jnp.full((L,), x, dtype=dtype)


def _sort_digit(v, shift):
  b = plsc.bitcast(v, jnp.int32)
  t = jnp.where(b < 0, ~b, b | _I32_MIN)
  key = ~t
  return (key >> shift) & (NRAD - 1)


def _body(nrows, vocab, logits_hbm, params_hbm, out_hbm,
          rowbuf, pbuf, hist, rhist, offs,
          cva, cia, cvb, cib, qsrc, gidx, zerobuf, sem):
  nv_row = vocab // L
  lane = lax.iota(jnp.int32, L)
  wid = lax.axis_index("s") * NC + lax.axis_index("c")

  ones_i = _vfull(1, jnp.int32)
  cal_cnt, _ = plsc.scan_count(jnp.zeros((L,), jnp.int32))
  sc_base = jnp.max(jnp.where(lane == 0, cal_cnt, 0))
  cs_probe = plsc.cumsum(ones_i)
  cs_incl = jnp.max(jnp.where(lane == 0, cs_probe, 0))

  def icumsum_i(x):
    return plsc.cumsum(x) + x * _vfull(1 - cs_incl, jnp.int32)

  def icumsum_f(x):
    adj = (jnp.int32(1) - cs_incl).astype(jnp.float32)
    return plsc.cumsum(x) + x * _vfull(adj, jnp.float32)

  def occ_rank(cnt):
    return cnt - _vfull(sc_base, jnp.int32)

  def occ_total(cnt):
    return cnt + _vfull(1 - sc_base, jnp.int32)

  def zb_fill(i, _):
    zerobuf[pl.ds(i * L, L)] = jnp.zeros((L,), jnp.float32)
    return 0
  lax.fori_loop(0, ZB // L, zb_fill, 0)

  def row_body(r, _):
    row = wid * nrows + r

    pltpu.sync_copy(logits_hbm.at[row], rowbuf)
    pltpu.sync_copy(params_hbm.at[row], pbuf)
    p_vec = pbuf[0]
    a_vec = pbuf[1]
    m_vec = pbuf[2]
    k_vec = pbuf[3].astype(jnp.int32)
    temp_vec = pbuf[4]

    def pa(i, carry):
      mxv, mnv = carry
      v = rowbuf[pl.ds(i * L, L)] / temp_vec
      rowbuf[pl.ds(i * L, L)] = v
      return jnp.maximum(mxv, v), jnp.minimum(mnv, v)
    mxv, mnv = lax.fori_loop(
        0, nv_row, pa, (_vfull(-jnp.inf), _vfull(jnp.inf)))
    mx = jnp.max(mxv)
    mn = jnp.min(mnv)
    mx_vec = _vfull(mx)
    scale_vec = _vfull(float(NBINS)) / jnp.maximum(
        mx_vec - _vfull(mn), _vfull(1e-30))

    def hz(i, _):
      hist[pl.ds(i * L, L)] = jnp.zeros((L,), jnp.int32)
      return 0
    lax.fori_loop(0, NBINS // L, hz, 0)

    def pb(i, acc):
      v = rowbuf[pl.ds(i * L, L)]
      e = jnp.exp(v - mx_vec)
      binf = (mx_vec - v) * scale_vec
      b = jnp.clip(binf.astype(jnp.int32), 0, NBINS - 1)
      cnt, last = plsc.scan_count(b)
      plsc.addupdate_scatter(hist, [b], occ_total(cnt), mask=last)
      return acc + e
    acc = lax.fori_loop(0, nv_row, pb, jnp.zeros((L,), jnp.float32))
    denom = jnp.sum(acc)
    d_vec = _vfull(denom)

    def pf(j, c):
      csum, bstar, found = c
      h = hist[pl.ds(j * L, L)]
      pref = _vfull(csum, jnp.int32) + icumsum_i(h)
      crossed = pref >= TOPK
      idxf = jnp.min(jnp.where(crossed, lane, 99))
      any_ = idxf < 99
      hit = jnp.logical_and(any_, found == 0)
      bstar = jnp.where(hit, j * L + idxf, bstar)
      return csum + jnp.sum(h), bstar, found | any_.astype(jnp.int32)
    _, bstar, _ = lax.fori_loop(
        0, NBINS // L, pf, (jnp.int32(0), jnp.int32(NBINS - 1), jnp.int32(0)))
    bstar_vec = _vfull(bstar, jnp.int32)

    def sf(i, _):
      cva[pl.ds(i * L, L)] = _vfull(_NEG_INF)
      cvb[pl.ds(i * L, L)] = _vfull(_NEG_INF)
      return 0
    lax.fori_loop(0, (CAP + L) // L, sf, 0)

    def pc(i, cnt):
      v = rowbuf[pl.ds(i * L, L)]
      b = jnp.clip(((mx_vec - v) * scale_vec).astype(jnp.int32), 0, NBINS - 1)
      sel = b <= bstar_vec
      base = jnp.minimum(cnt, CAP)
      plsc.store_compressed(cva.at[pl.ds(base, L)], v, mask=sel)
      idxv = _vfull(i * L, jnp.int32) + lane
      plsc.store_compressed(cia.at[pl.ds(base, L)], idxv, mask=sel)
      return cnt + jnp.sum(sel.astype(jnp.int32))
    cnt = lax.fori_loop(0, nv_row, pc, jnp.int32(0))
    nsel = jnp.minimum(cnt, jnp.int32(CAP))
    nvc = (nsel + L - 1) >> 4

    bufs = [(cva, cia), (cvb, cib)]
    for pidx in range(NPASS):
      vsrc, isrc = bufs[pidx % 2]
      vdst, idst = bufs[(pidx + 1) % 2]
      shift = RADIX_BITS * pidx

      def rz(i, _):
        rhist[pl.ds(i * L, L)] = jnp.zeros((L,), jnp.int32)
        return 0
      lax.fori_loop(0, NRAD // L, rz, 0)

      def h1(j, _, vsrc=vsrc, shift=shift):
        d = _sort_digit(vsrc[pl.ds(j * L, L)], shift)
        cnt1, last1 = plsc.scan_count(d)
        plsc.addupdate_scatter(rhist, [d], occ_total(cnt1), mask=last1)
        return 0
      lax.fori_loop(0, nvc, h1, 0)

      def h2(j, c):
        h = rhist[pl.ds(j * L, L)]
        inc = icumsum_i(h)
        offs[pl.ds(j * L, L)] = _vfull(c, jnp.int32) + inc - h
        return c + jnp.sum(h)
      lax.fori_loop(0, NRAD // L, h2, jnp.int32(0))

      def h3(j, _, vsrc=vsrc, isrc=isrc, vdst=vdst, idst=idst, shift=shift):
        v = vsrc[pl.ds(j * L, L)]
        iv = isrc[pl.ds(j * L, L)]
        d = _sort_digit(v, shift)
        cnt3, last3 = plsc.scan_count(d)
        basek = plsc.load_gather(offs, [d])
        pos = basek + occ_rank(cnt3)
        plsc.store_scatter(vdst, [pos], v)
        plsc.store_scatter(idst, [pos], iv)
        plsc.addupdate_scatter(offs, [d], occ_total(cnt3), mask=last3)
        return 0
      lax.fori_loop(0, nvc, h3, 0)

    q0 = _vfull(1.0) / d_vec
    t_vec = jnp.maximum(m_vec * q0, a_vec * q0 * q0)
    zero_v = jnp.zeros((L,), jnp.float32)

    def fm(j, c):
      csum, skeep = c
      v = cva[pl.ds(j * L, L)]
      q = jnp.exp(v - mx_vec) / d_vec
      incq = icumsum_f(q)
      excl = _vfull(csum) + incq - q
      ranks = _vfull(j * L, jnp.int32) + lane
      keep = (ranks < k_vec) & (
          (ranks == 0) | ((q >= t_vec) & (excl <= p_vec)))
      qk = jnp.where(keep, q, zero_v)
      jj = j >> 3
      col = (j & 7) * L
      qsrc[jj, pl.ds(col, L)] = qk
      gidx[jj, pl.ds(col, L)] = cia[pl.ds(j * L, L)] + _vfull(
          row * vocab, jnp.int32)
      return csum + jnp.sum(q), skeep + jnp.sum(qk)
    _, skeep = lax.fori_loop(
        0, TOPK // L, fm, (jnp.float32(0.0), jnp.float32(0.0)))
    skeep_vec = _vfull(skeep)

    def fd(j, _):
      jj = j >> 3
      col = (j & 7) * L
      qsrc[jj, pl.ds(col, L)] = qsrc[jj, pl.ds(col, L)] / skeep_vec
      return 0
    lax.fori_loop(0, TOPK // L, fd, 0)

    rbase = row * vocab
    nfull = vocab // ZB
    for c in range(nfull):
      pltpu.sync_copy(zerobuf, out_hbm.at[pl.ds(rbase + c * ZB, ZB)])
    tail = vocab - nfull * ZB
    if tail:
      pltpu.sync_copy(zerobuf.at[pl.ds(0, tail)],
                      out_hbm.at[pl.ds(rbase + nfull * ZB, tail)])

    for j in range(TOPK // 128):
      pltpu.async_copy(qsrc.at[j], out_hbm.at[gidx.at[j]], sem).wait()
    return 0

  lax.fori_loop(0, nrows, row_body, 0)


def kernel(logits, p, k, a, m, temperatures):
  b, v = logits.shape
  nrows = b // NW
  temps = jnp.where(temperatures == 0.0, 1.0, temperatures)
  params = jnp.stack(
      [p, a, m, k.astype(jnp.float32), temps], axis=1)
  params3 = jnp.broadcast_to(params[:, :, None], (b, 5, L))
  params3 = jnp.asarray(params3, jnp.float32)

  mesh = plsc.VectorSubcoreMesh(
      core_axis_name="c", subcore_axis_name="s",
      num_cores=NC, num_subcores=NS)
  run = pl.kernel(
      functools.partial(_body, nrows, v),
      out_type=jax.ShapeDtypeStruct((b * v,), jnp.float32),
      mesh=mesh,
      scratch_types=[
          pltpu.VMEM((v,), jnp.float32),
          pltpu.VMEM((5, L), jnp.float32),
          pltpu.VMEM((NBINS,), jnp.int32),
          pltpu.VMEM((NRAD,), jnp.int32),
          pltpu.VMEM((NRAD,), jnp.int32),
          pltpu.VMEM((CAP + L,), jnp.float32),
          pltpu.VMEM((CAP + L,), jnp.int32),
          pltpu.VMEM((CAP + L,), jnp.float32),
          pltpu.VMEM((CAP + L,), jnp.int32),
          pltpu.VMEM((TOPK // 128, 128), jnp.float32),
          pltpu.VMEM((TOPK // 128, 128), jnp.int32),
          pltpu.VMEM((ZB,), jnp.float32),
          pltpu.SemaphoreType.DMA,
      ],
      compiler_params=pltpu.CompilerParams(needs_layout_passes=False),
  )
  out_flat = run(logits, params3)
  return out_flat.reshape(b, v)


# speedup vs baseline: 26.4986x; 1.2934x over previous
import functools

import jax
import jax.numpy as jnp
from jax import lax
from jax.experimental import pallas as pl
from jax.experimental.pallas import tpu as pltpu
from jax.experimental.pallas import tpu_sc as plsc

L = 16
NC, NS = 2, 16
NW = NC * NS

NBINS = 256
CAP = 2048
TOPK = 1024
RADIX_BITS = 8
NRAD = 1 << RADIX_BITS
NPASS = 4
ZB = 16384

_I32_MIN = -2147483648
_NEG_INF = float("-inf")


def _vfull(x, dtype=jnp.float32):
  return jnp.full((L,), x, dtype=dtype)


def _sort_digit(v, shift):
  b = plsc.bitcast(v, jnp.int32)
  t = jnp.where(b < 0, ~b, b | _I32_MIN)
  key = ~t
  return (key >> shift) & (NRAD - 1)


def _body(nrows, vocab, logits_hbm, params_hbm, out_hbm,
          rowbuf, pbuf, hist, rhist, offs,
          cva, cia, cvb, cib, qsrc, gidx, zerobuf, sem):
  nv_row = vocab // L
  lane = lax.iota(jnp.int32, L)
  wid = lax.axis_index("s") * NC + lax.axis_index("c")

  ones_i = _vfull(1, jnp.int32)
  cal_cnt, _ = plsc.scan_count(jnp.zeros((L,), jnp.int32))
  sc_base = jnp.max(jnp.where(lane == 0, cal_cnt, 0))
  cs_probe = plsc.cumsum(ones_i)
  cs_incl = jnp.max(jnp.where(lane == 0, cs_probe, 0))

  def icumsum_i(x):
    return plsc.cumsum(x) + x * _vfull(1 - cs_incl, jnp.int32)

  def icumsum_f(x):
    adj = (jnp.int32(1) - cs_incl).astype(jnp.float32)
    return plsc.cumsum(x) + x * _vfull(adj, jnp.float32)

  def occ_rank(cnt):
    return cnt - _vfull(sc_base, jnp.int32)

  def occ_total(cnt):
    return cnt + _vfull(1 - sc_base, jnp.int32)

  def zb_fill(i, _):
    zerobuf[pl.ds(i * L, L)] = jnp.zeros((L,), jnp.float32)
    return 0
  lax.fori_loop(0, ZB // L, zb_fill, 0)

  def row_body(r, _):
    row = wid * nrows + r

    pltpu.sync_copy(logits_hbm.at[row], rowbuf)
    pltpu.sync_copy(params_hbm.at[row], pbuf)
    p_vec = pbuf[0]
    a_vec = pbuf[1]
    m_vec = pbuf[2]
    k_vec = pbuf[3].astype(jnp.int32)
    temp_vec = pbuf[4]

    UA = 10
    def pa2(i, carry):
      mxv, mnv = carry
      base = i * (UA * L)
      vs = [rowbuf[pl.ds(base + u * L, L)] for u in range(UA)]
      mxs, mns = list(vs), list(vs)
      while len(mxs) > 1:
        mxs = [jnp.maximum(mxs[t], mxs[t + 1]) for t in range(0, len(mxs) - 1, 2)] + (
            [mxs[-1]] if len(mxs) % 2 else [])
        mns = [jnp.minimum(mns[t], mns[t + 1]) for t in range(0, len(mns) - 1, 2)] + (
            [mns[-1]] if len(mns) % 2 else [])
      return jnp.maximum(mxv, mxs[0]), jnp.minimum(mnv, mns[0])
    mxv, mnv = lax.fori_loop(
        0, nv_row // UA, pa2, (_vfull(-jnp.inf), _vfull(jnp.inf)))
    mx = jnp.max(mxv)
    mn = jnp.min(mnv)
    mxr_vec = _vfull(mx)
    inv_temp = _vfull(1.0) / temp_vec
    mx_vec = mxr_vec / temp_vec
    scale_vec = _vfull(float(NBINS)) / jnp.maximum(
        mxr_vec - _vfull(mn), _vfull(1e-30))

    def hz(i, _):
      hist[pl.ds(i * L, L)] = jnp.zeros((L,), jnp.int32)
      return 0
    lax.fori_loop(0, NBINS // L, hz, 0)

    UB = 5
    def pb(i, carry):
      acc0, acc1 = carry
      base = i * (UB * L)
      accs = [acc0, acc1]
      for u in range(UB):
        v = rowbuf[pl.ds(base + u * L, L)]
        e = jnp.exp((v - mxr_vec) * inv_temp)
        accs[u & 1] = accs[u & 1] + e
        b = jnp.clip(((mxr_vec - v) * scale_vec).astype(jnp.int32),
                     0, NBINS - 1)
        cntb, lastb = plsc.scan_count(b)
        plsc.addupdate_scatter(hist, [b], occ_total(cntb), mask=lastb)
      return accs[0], accs[1]
    acc0, acc1 = lax.fori_loop(
        0, nv_row // UB, pb,
        (jnp.zeros((L,), jnp.float32), jnp.zeros((L,), jnp.float32)))
    denom = jnp.sum(acc0 + acc1)
    d_vec = _vfull(denom)

    def pf(j, c):
      csum, bstar, found = c
      h = hist[pl.ds(j * L, L)]
      pref = _vfull(csum, jnp.int32) + icumsum_i(h)
      crossed = pref >= TOPK
      idxf = jnp.min(jnp.where(crossed, lane, 99))
      any_ = idxf < 99
      hit = jnp.logical_and(any_, found == 0)
      bstar = jnp.where(hit, j * L + idxf, bstar)
      return csum + jnp.sum(h), bstar, found | any_.astype(jnp.int32)
    _, bstar, _ = lax.fori_loop(
        0, NBINS // L, pf, (jnp.int32(0), jnp.int32(NBINS - 1), jnp.int32(0)))
    bstar_vec = _vfull(bstar, jnp.int32)

    def sf(i, _):
      cva[pl.ds(i * L, L)] = _vfull(_NEG_INF)
      cvb[pl.ds(i * L, L)] = _vfull(_NEG_INF)
      return 0
    lax.fori_loop(0, (CAP + L) // L, sf, 0)

    UC = 5
    def pc(i, cnt):
      base = i * (UC * L)
      vs, sels, sums = [], [], []
      for u in range(UC):
        v = rowbuf[pl.ds(base + u * L, L)]
        b = jnp.clip(((mxr_vec - v) * scale_vec).astype(jnp.int32),
                     0, NBINS - 1)
        sel = b <= bstar_vec
        vs.append(v); sels.append(sel)
        sums.append(jnp.sum(sel.astype(jnp.int32)))
      for u in range(UC):
        off = jnp.minimum(cnt, CAP)
        plsc.store_compressed(cva.at[pl.ds(off, L)], vs[u], mask=sels[u])
        idxv = _vfull(base + u * L, jnp.int32) + lane
        plsc.store_compressed(cia.at[pl.ds(off, L)], idxv, mask=sels[u])
        cnt = cnt + sums[u]
      return cnt
    cnt = lax.fori_loop(0, nv_row // UC, pc, jnp.int32(0))
    nsel = jnp.minimum(cnt, jnp.int32(CAP))
    nvc = (nsel + L - 1) >> 4

    def csc(j, _):
      cva[pl.ds(j * L, L)] = cva[pl.ds(j * L, L)] / temp_vec
      return 0
    lax.fori_loop(0, nvc, csc, 0)

    bufs = [(cva, cia), (cvb, cib)]
    for pidx in range(NPASS):
      vsrc, isrc = bufs[pidx % 2]
      vdst, idst = bufs[(pidx + 1) % 2]
      shift = RADIX_BITS * pidx

      def rz(i, _):
        rhist[pl.ds(i * L, L)] = jnp.zeros((L,), jnp.int32)
        return 0
      lax.fori_loop(0, NRAD // L, rz, 0)

      def h1(j, _, vsrc=vsrc, shift=shift):
        d = _sort_digit(vsrc[pl.ds(j * L, L)], shift)
        cnt1, last1 = plsc.scan_count(d)
        plsc.addupdate_scatter(rhist, [d], occ_total(cnt1), mask=last1)
        return 0
      lax.fori_loop(0, nvc, h1, 0)

      def h2(j, c):
        h = rhist[pl.ds(j * L, L)]
        inc = icumsum_i(h)
        offs[pl.ds(j * L, L)] = _vfull(c, jnp.int32) + inc - h
        return c + jnp.sum(h)
      lax.fori_loop(0, NRAD // L, h2, jnp.int32(0))

      def h3(j, _, vsrc=vsrc, isrc=isrc, vdst=vdst, idst=idst, shift=shift):
        v = vsrc[pl.ds(j * L, L)]
        iv = isrc[pl.ds(j * L, L)]
        d = _sort_digit(v, shift)
        cnt3, last3 = plsc.scan_count(d)
        basek = plsc.load_gather(offs, [d])
        pos = basek + occ_rank(cnt3)
        plsc.store_scatter(vdst, [pos], v)
        plsc.store_scatter(idst, [pos], iv)
        plsc.addupdate_scatter(offs, [d], occ_total(cnt3), mask=last3)
        return 0
      lax.fori_loop(0, nvc, h3, 0)

    q0 = _vfull(1.0) / d_vec
    t_vec = jnp.maximum(m_vec * q0, a_vec * q0 * q0)
    zero_v = jnp.zeros((L,), jnp.float32)

    def fm(j, c):
      csum, skeep = c
      v = cva[pl.ds(j * L, L)]
      q = jnp.exp(v - mx_vec) / d_vec
      incq = icumsum_f(q)
      excl = _vfull(csum) + incq - q
      ranks = _vfull(j * L, jnp.int32) + lane
      keep = (ranks < k_vec) & (
          (ranks == 0) | ((q >= t_vec) & (excl <= p_vec)))
      qk = jnp.where(keep, q, zero_v)
      jj = j >> 3
      col = (j & 7) * L
      qsrc[jj, pl.ds(col, L)] = qk
      gidx[jj, pl.ds(col, L)] = cia[pl.ds(j * L, L)] + _vfull(
          row * vocab, jnp.int32)
      return csum + jnp.sum(q), skeep + jnp.sum(qk)
    _, skeep = lax.fori_loop(
        0, TOPK // L, fm, (jnp.float32(0.0), jnp.float32(0.0)))
    skeep_vec = _vfull(skeep)

    def fd(j, _):
      jj = j >> 3
      col = (j & 7) * L
      qsrc[jj, pl.ds(col, L)] = qsrc[jj, pl.ds(col, L)] / skeep_vec
      return 0
    lax.fori_loop(0, TOPK // L, fd, 0)

    rbase = row * vocab
    nfull = vocab // ZB
    for c in range(nfull):
      pltpu.sync_copy(zerobuf, out_hbm.at[pl.ds(rbase + c * ZB, ZB)])
    tail = vocab - nfull * ZB
    if tail:
      pltpu.sync_copy(zerobuf.at[pl.ds(0, tail)],
                      out_hbm.at[pl.ds(rbase + nfull * ZB, tail)])

    for j in range(TOPK // 128):
      pltpu.async_copy(qsrc.at[j], out_hbm.at[gidx.at[j]], sem).wait()
    return 0

  lax.fori_loop(0, nrows, row_body, 0)


def kernel(logits, p, k, a, m, temperatures):
  b, v = logits.shape
  nrows = b // NW
  temps = jnp.where(temperatures == 0.0, 1.0, temperatures)
  params = jnp.stack(
      [p, a, m, k.astype(jnp.float32), temps], axis=1)
  params3 = jnp.broadcast_to(params[:, :, None], (b, 5, L))
  params3 = jnp.asarray(params3, jnp.float32)

  mesh = plsc.VectorSubcoreMesh(
      core_axis_name="c", subcore_axis_name="s",
      num_cores=NC, num_subcores=NS)
  run = pl.kernel(
      functools.partial(_body, nrows, v),
      out_type=jax.ShapeDtypeStruct((b * v,), jnp.float32),
      mesh=mesh,
      scratch_types=[
          pltpu.VMEM((v,), jnp.float32),
          pltpu.VMEM((5, L), jnp.float32),
          pltpu.VMEM((NBINS,), jnp.int32),
          pltpu.VMEM((NRAD,), jnp.int32),
          pltpu.VMEM((NRAD,), jnp.int32),
          pltpu.VMEM((CAP + L,), jnp.float32),
          pltpu.VMEM((CAP + L,), jnp.int32),
          pltpu.VMEM((CAP + L,), jnp.float32),
          pltpu.VMEM((CAP + L,), jnp.int32),
          pltpu.VMEM((TOPK // 128, 128), jnp.float32),
          pltpu.VMEM((TOPK // 128, 128), jnp.int32),
          pltpu.VMEM((ZB,), jnp.float32),
          pltpu.SemaphoreType.DMA,
      ],
      compiler_params=pltpu.CompilerParams(needs_layout_passes=False),
  )
  out_flat = run(logits, params3)
  return out_flat.reshape(b, v)


# speedup vs baseline: 34.6734x; 1.3085x over previous
import functools

import jax
import jax.numpy as jnp
from jax import lax
from jax.experimental import pallas as pl
from jax.experimental.pallas import tpu as pltpu
from jax.experimental.pallas import tpu_sc as plsc

L = 16
NC, NS = 2, 16
NW = NC * NS

NBINS = 128
CAP = 2048
TOPK = 1024
RADIX_BITS = 8
NRAD = 1 << RADIX_BITS
NPASS = 4
ZB = 8192

_I32_MIN = -2147483648
_NEG_INF = float("-inf")


def _vfull(x, dtype=jnp.float32):
  return jnp.full((L,), x, dtype=dtype)


def _sort_digit(v, shift):
  b = plsc.bitcast(v, jnp.int32)
  t = jnp.where(b < 0, ~b, b | _I32_MIN)
  key = ~t
  return (key >> shift) & (NRAD - 1)


def _body(nrows, vocab, logits_hbm, params_hbm, out_hbm,
          rowbuf, pbuf, hist, rhist, offs,
          cva, cia, cvb, cib, qsrc, gidx, zerobuf, sem):
  nv_row = vocab // L
  lane = lax.iota(jnp.int32, L)
  wid = lax.axis_index("s") * NC + lax.axis_index("c")

  ones_i = _vfull(1, jnp.int32)
  cal_cnt, _ = plsc.scan_count(jnp.zeros((L,), jnp.int32))
  sc_base = jnp.max(jnp.where(lane == 0, cal_cnt, 0))
  cs_probe = plsc.cumsum(ones_i)
  cs_incl = jnp.max(jnp.where(lane == 0, cs_probe, 0))

  def icumsum_i(x):
    return plsc.cumsum(x) + x * _vfull(1 - cs_incl, jnp.int32)

  def icumsum_f(x):
    adj = (jnp.int32(1) - cs_incl).astype(jnp.float32)
    return plsc.cumsum(x) + x * _vfull(adj, jnp.float32)

  def occ_rank(cnt):
    return cnt - _vfull(sc_base, jnp.int32)

  def occ_total(cnt):
    return cnt + _vfull(1 - sc_base, jnp.int32)

  def zb_fill(i, _):
    zerobuf[pl.ds(i * L, L)] = jnp.zeros((L,), jnp.float32)
    return 0
  lax.fori_loop(0, ZB // L, zb_fill, 0)

  def row_body(r, _):
    row = wid * nrows + r

    pltpu.sync_copy(logits_hbm.at[row], rowbuf)
    pltpu.sync_copy(params_hbm.at[row], pbuf)
    p_vec = pbuf[0]
    a_vec = pbuf[1]
    m_vec = pbuf[2]
    k_vec = pbuf[3].astype(jnp.int32)
    temp_vec = pbuf[4]

    UA = 10
    def pa2(i, carry):
      mxv, mnv = carry
      base = i * (UA * L)
      vs = [rowbuf[pl.ds(base + u * L, L)] for u in range(UA)]
      mxs, mns = list(vs), list(vs)
      while len(mxs) > 1:
        mxs = [jnp.maximum(mxs[t], mxs[t + 1]) for t in range(0, len(mxs) - 1, 2)] + (
            [mxs[-1]] if len(mxs) % 2 else [])
        mns = [jnp.minimum(mns[t], mns[t + 1]) for t in range(0, len(mns) - 1, 2)] + (
            [mns[-1]] if len(mns) % 2 else [])
      return jnp.maximum(mxv, mxs[0]), jnp.minimum(mnv, mns[0])
    mxv, mnv = lax.fori_loop(
        0, nv_row // UA, pa2, (_vfull(-jnp.inf), _vfull(jnp.inf)))
    mx = jnp.max(mxv)
    mn = jnp.min(mnv)
    mxr_vec = _vfull(mx)
    inv_temp = _vfull(1.0) / temp_vec
    mx_vec = mxr_vec / temp_vec
    scale_vec = _vfull(float(NBINS)) / jnp.maximum(
        mxr_vec - _vfull(mn), _vfull(1e-30))

    def hz(i, _):
      hist[pl.ds(i * L, L)] = jnp.zeros((L,), jnp.int32)
      return 0
    lax.fori_loop(0, NBINS, hz, 0)

    UB = 5
    ones_i32 = _vfull(1, jnp.int32)
    def pb(i, carry):
      acc0, acc1 = carry
      base = i * (UB * L)
      accs = [acc0, acc1]
      for u in range(UB):
        v = rowbuf[pl.ds(base + u * L, L)]
        e = jnp.exp((v - mxr_vec) * inv_temp)
        accs[u & 1] = accs[u & 1] + e
        b = jnp.clip(((mxr_vec - v) * scale_vec).astype(jnp.int32),
                     0, NBINS - 1)
        plsc.addupdate_scatter(hist, [(b << 4) | lane], ones_i32)
      return accs[0], accs[1]
    acc0, acc1 = lax.fori_loop(
        0, nv_row // UB, pb,
        (jnp.zeros((L,), jnp.float32), jnp.zeros((L,), jnp.float32)))
    denom = jnp.sum(acc0 + acc1)
    d_vec = _vfull(denom)

    def pf(b, c):
      csum, bstar, found = c
      tot = jnp.sum(hist[pl.ds(b * L, L)])
      csum2 = csum + tot
      hit = jnp.logical_and(csum2 >= TOPK, found == 0)
      bstar = jnp.where(hit, b, bstar)
      return csum2, bstar, found | hit.astype(jnp.int32)
    _, bstar, _ = lax.fori_loop(
        0, NBINS, pf, (jnp.int32(0), jnp.int32(NBINS - 1), jnp.int32(0)))
    range_vec = jnp.maximum(mxr_vec - _vfull(mn), _vfull(1e-30))
    tstar_vec = mxr_vec - (
        (_vfull(bstar.astype(jnp.float32)) + _vfull(1.5))
        * range_vec * _vfull(1.0 / NBINS))

    def sf(i, _):
      cva[pl.ds(i * L, L)] = _vfull(_NEG_INF)
      cvb[pl.ds(i * L, L)] = _vfull(_NEG_INF)
      return 0
    lax.fori_loop(0, (CAP + L) // L, sf, 0)

    UC = 5
    def pc(i, cnt):
      base = i * (UC * L)
      vs, sels, sums = [], [], []
      for u in range(UC):
        v = rowbuf[pl.ds(base + u * L, L)]
        sel = v >= tstar_vec
        vs.append(v); sels.append(sel)
        sums.append(jnp.sum(sel.astype(jnp.int32)))
      for u in range(UC):
        off = jnp.minimum(cnt, CAP)
        plsc.store_compressed(cva.at[pl.ds(off, L)], vs[u], mask=sels[u])
        idxv = _vfull(base + u * L, jnp.int32) + lane
        plsc.store_compressed(cia.at[pl.ds(off, L)], idxv, mask=sels[u])
        cnt = cnt + sums[u]
      return cnt
    cnt = lax.fori_loop(0, nv_row // UC, pc, jnp.int32(0))
    nsel = jnp.minimum(cnt, jnp.int32(CAP))
    nvc = (nsel + L - 1) >> 4

    def csc(j, _):
      cva[pl.ds(j * L, L)] = cva[pl.ds(j * L, L)] / temp_vec
      return 0
    lax.fori_loop(0, nvc, csc, 0)

    bufs = [(cva, cia), (cvb, cib)]
    for pidx in range(NPASS):
      vsrc, isrc = bufs[pidx % 2]
      vdst, idst = bufs[(pidx + 1) % 2]
      shift = RADIX_BITS * pidx

      def rz(i, _):
        rhist[pl.ds(i * L, L)] = jnp.zeros((L,), jnp.int32)
        return 0
      lax.fori_loop(0, NRAD // L, rz, 0)

      def h1(j, _, vsrc=vsrc, shift=shift):
        d = _sort_digit(vsrc[pl.ds(j * L, L)], shift)
        cnt1, last1 = plsc.scan_count(d)
        plsc.addupdate_scatter(rhist, [d], occ_total(cnt1), mask=last1)
        return 0
      lax.fori_loop(0, nvc, h1, 0)

      def h2(j, c):
        h = rhist[pl.ds(j * L, L)]
        inc = icumsum_i(h)
        offs[pl.ds(j * L, L)] = _vfull(c, jnp.int32) + inc - h
        return c + jnp.sum(h)
      lax.fori_loop(0, NRAD // L, h2, jnp.int32(0))

      def h3(j, _, vsrc=vsrc, isrc=isrc, vdst=vdst, idst=idst, shift=shift):
        v = vsrc[pl.ds(j * L, L)]
        iv = isrc[pl.ds(j * L, L)]
        d = _sort_digit(v, shift)
        cnt3, last3 = plsc.scan_count(d)
        basek = plsc.load_gather(offs, [d])
        pos = basek + occ_rank(cnt3)
        plsc.store_scatter(vdst, [pos], v)
        plsc.store_scatter(idst, [pos], iv)
        plsc.addupdate_scatter(offs, [d], occ_total(cnt3), mask=last3)
        return 0
      lax.fori_loop(0, nvc, h3, 0)

    q0 = _vfull(1.0) / d_vec
    t_vec = jnp.maximum(m_vec * q0, a_vec * q0 * q0)
    zero_v = jnp.zeros((L,), jnp.float32)

    def fm(j, c):
      csum, skeep = c
      v = cva[pl.ds(j * L, L)]
      q = jnp.exp(v - mx_vec) / d_vec
      incq = icumsum_f(q)
      excl = _vfull(csum) + incq - q
      ranks = _vfull(j * L, jnp.int32) + lane
      keep = (ranks < k_vec) & (
          (ranks == 0) | ((q >= t_vec) & (excl <= p_vec)))
      qk = jnp.where(keep, q, zero_v)
      jj = j >> 3
      col = (j & 7) * L
      qsrc[jj, pl.ds(col, L)] = qk
      gidx[jj, pl.ds(col, L)] = cia[pl.ds(j * L, L)] + _vfull(
          row * vocab, jnp.int32)
      return csum + jnp.sum(q), skeep + jnp.sum(qk)
    _, skeep = lax.fori_loop(
        0, TOPK // L, fm, (jnp.float32(0.0), jnp.float32(0.0)))
    skeep_vec = _vfull(skeep)

    def fd(j, _):
      jj = j >> 3
      col = (j & 7) * L
      qsrc[jj, pl.ds(col, L)] = qsrc[jj, pl.ds(col, L)] / skeep_vec
      return 0
    lax.fori_loop(0, TOPK // L, fd, 0)

    rbase = row * vocab
    nfull = vocab // ZB
    for c in range(nfull):
      pltpu.sync_copy(zerobuf, out_hbm.at[pl.ds(rbase + c * ZB, ZB)])
    tail = vocab - nfull * ZB
    if tail:
      pltpu.sync_copy(zerobuf.at[pl.ds(0, tail)],
                      out_hbm.at[pl.ds(rbase + nfull * ZB, tail)])

    for j in range(TOPK // 128):
      pltpu.async_copy(qsrc.at[j], out_hbm.at[gidx.at[j]], sem).wait()
    return 0

  lax.fori_loop(0, nrows, row_body, 0)


def kernel(logits, p, k, a, m, temperatures):
  b, v = logits.shape
  nrows = b // NW
  temps = jnp.where(temperatures == 0.0, 1.0, temperatures)
  params = jnp.stack(
      [p, a, m, k.astype(jnp.float32), temps], axis=1)
  params3 = jnp.broadcast_to(params[:, :, None], (b, 5, L))
  params3 = jnp.asarray(params3, jnp.float32)

  mesh = plsc.VectorSubcoreMesh(
      core_axis_name="c", subcore_axis_name="s",
      num_cores=NC, num_subcores=NS)
  run = pl.kernel(
      functools.partial(_body, nrows, v),
      out_type=jax.ShapeDtypeStruct((b * v,), jnp.float32),
      mesh=mesh,
      scratch_types=[
          pltpu.VMEM((v,), jnp.float32),
          pltpu.VMEM((5, L), jnp.float32),
          pltpu.VMEM((NBINS * L,), jnp.int32),
          pltpu.VMEM((NRAD,), jnp.int32),
          pltpu.VMEM((NRAD,), jnp.int32),
          pltpu.VMEM((CAP + L,), jnp.float32),
          pltpu.VMEM((CAP + L,), jnp.int32),
          pltpu.VMEM((CAP + L,), jnp.float32),
          pltpu.VMEM((CAP + L,), jnp.int32),
          pltpu.VMEM((TOPK // 128, 128), jnp.float32),
          pltpu.VMEM((TOPK // 128, 128), jnp.int32),
          pltpu.VMEM((ZB,), jnp.float32),
          pltpu.SemaphoreType.DMA,
      ],
      compiler_params=pltpu.CompilerParams(needs_layout_passes=False),
  )
  out_flat = run(logits, params3)
  return out_flat.reshape(b, v)


# speedup vs baseline: 58.4004x; 1.6843x over previous
import functools

import jax
import jax.numpy as jnp
from jax import lax
from jax.experimental import pallas as pl
from jax.experimental.pallas import tpu as pltpu
from jax.experimental.pallas import tpu_sc as plsc

L = 16
NC, NS = 2, 16
NW = NC * NS

NBINS = 128
CAP = 2048
TOPK = 1024
RADIX_BITS = 8
NRAD = 1 << RADIX_BITS
NPASS = 4
ZB = 8192

_I32_MIN = -2147483648
_NEG_INF = float("-inf")


def _vfull(x, dtype=jnp.float32):
  return jnp.full((L,), x, dtype=dtype)


def _sort_digit(v, shift):
  b = plsc.bitcast(v, jnp.int32)
  t = jnp.where(b < 0, ~b, b | _I32_MIN)
  key = ~t
  return (key >> shift) & (NRAD - 1)


def _body(nrows, vocab, logits_hbm, params_hbm, out_hbm,
          rowbuf, pbuf, hist, rhist, offs,
          cva, cia, cvb, cib, qsrc, gidx, zerobuf, sem):
  nv_row = vocab // L
  lane = lax.iota(jnp.int32, L)
  wid = lax.axis_index("s") * NC + lax.axis_index("c")

  ones_i = _vfull(1, jnp.int32)
  cal_cnt, _ = plsc.scan_count(jnp.zeros((L,), jnp.int32))
  sc_base = jnp.max(jnp.where(lane == 0, cal_cnt, 0))
  cs_probe = plsc.cumsum(ones_i)
  cs_incl = jnp.max(jnp.where(lane == 0, cs_probe, 0))

  def icumsum_i(x):
    return plsc.cumsum(x) + x * _vfull(1 - cs_incl, jnp.int32)

  def icumsum_f(x):
    adj = (jnp.int32(1) - cs_incl).astype(jnp.float32)
    return plsc.cumsum(x) + x * _vfull(adj, jnp.float32)

  def occ_rank(cnt):
    return cnt - _vfull(sc_base, jnp.int32)

  def occ_total(cnt):
    return cnt + _vfull(1 - sc_base, jnp.int32)

  def zb_fill(i, _):
    zerobuf[pl.ds(i * L, L)] = jnp.zeros((L,), jnp.float32)
    return 0
  lax.fori_loop(0, ZB // L, zb_fill, 0)

  def row_body(r, _):
    row = wid * nrows + r

    pltpu.sync_copy(logits_hbm.at[row], rowbuf)
    pltpu.sync_copy(params_hbm.at[row], pbuf)
    p_vec = pbuf[0]
    a_vec = pbuf[1]
    m_vec = pbuf[2]
    k_vec = pbuf[3].astype(jnp.int32)
    temp_vec = pbuf[4]

    UA = 10
    def pa2(i, carry):
      mxv, mnv = carry
      base = i * (UA * L)
      vs = [rowbuf[pl.ds(base + u * L, L)] for u in range(UA)]
      mxs, mns = list(vs), list(vs)
      while len(mxs) > 1:
        mxs = [jnp.maximum(mxs[t], mxs[t + 1]) for t in range(0, len(mxs) - 1, 2)] + (
            [mxs[-1]] if len(mxs) % 2 else [])
        mns = [jnp.minimum(mns[t], mns[t + 1]) for t in range(0, len(mns) - 1, 2)] + (
            [mns[-1]] if len(mns) % 2 else [])
      return jnp.maximum(mxv, mxs[0]), jnp.minimum(mnv, mns[0])
    mxv, mnv = lax.fori_loop(
        0, nv_row // UA, pa2, (_vfull(-jnp.inf), _vfull(jnp.inf)))
    mx = jnp.max(mxv)
    mn = jnp.min(mnv)
    mxr_vec = _vfull(mx)
    inv_temp = _vfull(1.0) / temp_vec
    mx_vec = mxr_vec / temp_vec
    scale_vec = _vfull(float(NBINS)) / jnp.maximum(
        mxr_vec - _vfull(mn), _vfull(1e-30))

    def hz(i, _):
      hist[pl.ds(i * L, L)] = jnp.zeros((L,), jnp.int32)
      return 0
    lax.fori_loop(0, NBINS, hz, 0)

    UB = 10
    ones_i32 = _vfull(1, jnp.int32)
    def pb(i, acc):
      base = i * (UB * L)
      vs = [rowbuf[pl.ds(base + u * L, L)] for u in range(UB)]
      es = [jnp.exp((v - mxr_vec) * inv_temp) for v in vs]
      ixs = [((((mxr_vec - v) * scale_vec).astype(jnp.int32)) << 4) | lane
             for v in vs]
      for u in range(UB):
        plsc.addupdate_scatter(hist, [ixs[u]], ones_i32)
      while len(es) > 1:
        es = [es[t] + es[t + 1] for t in range(0, len(es) - 1, 2)] + (
            [es[-1]] if len(es) % 2 else [])
      return acc + es[0]
    acc = lax.fori_loop(0, nv_row // UB, pb, jnp.zeros((L,), jnp.float32))
    denom = jnp.sum(acc)
    d_vec = _vfull(denom)

    def pf(b, c):
      csum, bstar, found = c
      tot = jnp.sum(hist[pl.ds(b * L, L)])
      csum2 = csum + tot
      hit = jnp.logical_and(csum2 >= TOPK, found == 0)
      bstar = jnp.where(hit, b, bstar)
      return csum2, bstar, found | hit.astype(jnp.int32)
    _, bstar, _ = lax.fori_loop(
        0, NBINS, pf, (jnp.int32(0), jnp.int32(NBINS - 1), jnp.int32(0)))
    range_vec = jnp.maximum(mxr_vec - _vfull(mn), _vfull(1e-30))
    tstar_vec = mxr_vec - (
        (_vfull(bstar.astype(jnp.float32)) + _vfull(1.5))
        * range_vec * _vfull(1.0 / NBINS))

    def sf(i, _):
      cva[pl.ds(i * L, L)] = _vfull(_NEG_INF)
      cvb[pl.ds(i * L, L)] = _vfull(_NEG_INF)
      return 0
    lax.fori_loop(0, (CAP + L) // L, sf, 0)

    UC = 10
    def pc(i, cnt):
      base = i * (UC * L)
      vs, sels, sums = [], [], []
      for u in range(UC):
        v = rowbuf[pl.ds(base + u * L, L)]
        sel = v >= tstar_vec
        vs.append(v); sels.append(sel)
        sums.append(jnp.sum(sel.astype(jnp.int32)))
      for u in range(UC):
        off = jnp.minimum(cnt, CAP)
        plsc.store_compressed(cva.at[pl.ds(off, L)], vs[u], mask=sels[u])
        idxv = _vfull(base + u * L, jnp.int32) + lane
        plsc.store_compressed(cia.at[pl.ds(off, L)], idxv, mask=sels[u])
        cnt = cnt + sums[u]
      return cnt
    cnt = lax.fori_loop(0, nv_row // UC, pc, jnp.int32(0))
    nsel = jnp.minimum(cnt, jnp.int32(CAP))
    nvc = (nsel + L - 1) >> 4

    def csc(j, _):
      cva[pl.ds(j * L, L)] = cva[pl.ds(j * L, L)] / temp_vec
      return 0
    lax.fori_loop(0, nvc, csc, 0)

    bufs = [(cva, cia), (cvb, cib)]
    for pidx in range(NPASS):
      vsrc, isrc = bufs[pidx % 2]
      vdst, idst = bufs[(pidx + 1) % 2]
      shift = RADIX_BITS * pidx

      def rz(i, _):
        rhist[pl.ds(i * L, L)] = jnp.zeros((L,), jnp.int32)
        return 0
      lax.fori_loop(0, NRAD // L, rz, 0)

      def h1(j, _, vsrc=vsrc, shift=shift):
        d = _sort_digit(vsrc[pl.ds(j * L, L)], shift)
        cnt1, last1 = plsc.scan_count(d)
        plsc.addupdate_scatter(rhist, [d], occ_total(cnt1), mask=last1)
        return 0
      lax.fori_loop(0, nvc, h1, 0)

      def h2(j, c):
        h = rhist[pl.ds(j * L, L)]
        inc = icumsum_i(h)
        offs[pl.ds(j * L, L)] = _vfull(c, jnp.int32) + inc - h
        return c + jnp.sum(h)
      lax.fori_loop(0, NRAD // L, h2, jnp.int32(0))

      def h3(j, _, vsrc=vsrc, isrc=isrc, vdst=vdst, idst=idst, shift=shift):
        v = vsrc[pl.ds(j * L, L)]
        iv = isrc[pl.ds(j * L, L)]
        d = _sort_digit(v, shift)
        cnt3, last3 = plsc.scan_count(d)
        basek = plsc.load_gather(offs, [d])
        pos = basek + occ_rank(cnt3)
        plsc.store_scatter(vdst, [pos], v)
        plsc.store_scatter(idst, [pos], iv)
        plsc.addupdate_scatter(offs, [d], occ_total(cnt3), mask=last3)
        return 0
      lax.fori_loop(0, nvc, h3, 0)

    q0 = _vfull(1.0) / d_vec
    t_vec = jnp.maximum(m_vec * q0, a_vec * q0 * q0)
    zero_v = jnp.zeros((L,), jnp.float32)

    def fm(j, c):
      csum, skeep = c
      v = cva[pl.ds(j * L, L)]
      q = jnp.exp(v - mx_vec) / d_vec
      incq = icumsum_f(q)
      excl = _vfull(csum) + incq - q
      ranks = _vfull(j * L, jnp.int32) + lane
      keep = (ranks < k_vec) & (
          (ranks == 0) | ((q >= t_vec) & (excl <= p_vec)))
      qk = jnp.where(keep, q, zero_v)
      jj = j >> 3
      col = (j & 7) * L
      qsrc[jj, pl.ds(col, L)] = qk
      gidx[jj, pl.ds(col, L)] = cia[pl.ds(j * L, L)] + _vfull(
          row * vocab, jnp.int32)
      return csum + jnp.sum(q), skeep + jnp.sum(qk)
    _, skeep = lax.fori_loop(
        0, TOPK // L, fm, (jnp.float32(0.0), jnp.float32(0.0)))
    skeep_vec = _vfull(skeep)

    def fd(j, _):
      jj = j >> 3
      col = (j & 7) * L
      qsrc[jj, pl.ds(col, L)] = qsrc[jj, pl.ds(col, L)] / skeep_vec
      return 0
    lax.fori_loop(0, TOPK // L, fd, 0)

    rbase = row * vocab
    nfull = vocab // ZB
    for c in range(nfull):
      pltpu.sync_copy(zerobuf, out_hbm.at[pl.ds(rbase + c * ZB, ZB)])
    tail = vocab - nfull * ZB
    if tail:
      pltpu.sync_copy(zerobuf.at[pl.ds(0, tail)],
                      out_hbm.at[pl.ds(rbase + nfull * ZB, tail)])

    for j in range(TOPK // 128):
      pltpu.async_copy(qsrc.at[j], out_hbm.at[gidx.at[j]], sem).wait()
    return 0

  lax.fori_loop(0, nrows, row_body, 0)


def kernel(logits, p, k, a, m, temperatures):
  b, v = logits.shape
  nrows = b // NW
  temps = jnp.where(temperatures == 0.0, 1.0, temperatures)
  params = jnp.stack(
      [p, a, m, k.astype(jnp.float32), temps], axis=1)
  params3 = jnp.broadcast_to(params[:, :, None], (b, 5, L))
  params3 = jnp.asarray(params3, jnp.float32)

  mesh = plsc.VectorSubcoreMesh(
      core_axis_name="c", subcore_axis_name="s",
      num_cores=NC, num_subcores=NS)
  run = pl.kernel(
      functools.partial(_body, nrows, v),
      out_type=jax.ShapeDtypeStruct((b * v,), jnp.float32),
      mesh=mesh,
      scratch_types=[
          pltpu.VMEM((v,), jnp.float32),
          pltpu.VMEM((5, L), jnp.float32),
          pltpu.VMEM((NBINS * L + 2 * L,), jnp.int32),
          pltpu.VMEM((NRAD,), jnp.int32),
          pltpu.VMEM((NRAD,), jnp.int32),
          pltpu.VMEM((CAP + L,), jnp.float32),
          pltpu.VMEM((CAP + L,), jnp.int32),
          pltpu.VMEM((CAP + L,), jnp.float32),
          pltpu.VMEM((CAP + L,), jnp.int32),
          pltpu.VMEM((TOPK // 128, 128), jnp.float32),
          pltpu.VMEM((TOPK // 128, 128), jnp.int32),
          pltpu.VMEM((ZB,), jnp.float32),
          pltpu.SemaphoreType.DMA,
      ],
      compiler_params=pltpu.CompilerParams(needs_layout_passes=False),
  )
  out_flat = run(logits, params3)
  return out_flat.reshape(b, v)


# speedup vs baseline: 62.5532x; 1.0711x over previous
import functools

import jax
import jax.numpy as jnp
from jax import lax
from jax.experimental import pallas as pl
from jax.experimental.pallas import tpu as pltpu
from jax.experimental.pallas import tpu_sc as plsc

L = 16
NC, NS = 2, 16
NW = NC * NS

NBINS = 128
CAP = 2048
TOPK = 1024
RADIX_BITS = 8
NRAD = 1 << RADIX_BITS
NPASS = 4
ZB = 8192

_I32_MIN = -2147483648
_NEG_INF = float("-inf")


def _vfull(x, dtype=jnp.float32):
  return jnp.full((L,), x, dtype=dtype)


def _sort_digit(v, shift):
  b = plsc.bitcast(v, jnp.int32)
  t = jnp.where(b < 0, ~b, b | _I32_MIN)
  key = ~t
  return (key >> shift) & (NRAD - 1)


def _body(nrows, vocab, logits_hbm, params_hbm, out_hbm,
          rowbuf, pbuf, hist, rhist, offs,
          cva, cia, cvb, cib, qsrc, gidx, zerobuf,
          sem_s, sem_z, sem_d):
  nv_row = vocab // L
  lane = lax.iota(jnp.int32, L)
  wid = lax.axis_index("s") * NC + lax.axis_index("c")

  ones_i = _vfull(1, jnp.int32)
  cal_cnt, _ = plsc.scan_count(jnp.zeros((L,), jnp.int32))
  sc_base = jnp.max(jnp.where(lane == 0, cal_cnt, 0))
  cs_probe = plsc.cumsum(ones_i)
  cs_incl = jnp.max(jnp.where(lane == 0, cs_probe, 0))

  def icumsum_i(x):
    return plsc.cumsum(x) + x * _vfull(1 - cs_incl, jnp.int32)

  def icumsum_f(x):
    adj = (jnp.int32(1) - cs_incl).astype(jnp.float32)
    return plsc.cumsum(x) + x * _vfull(adj, jnp.float32)

  def occ_rank(cnt):
    return cnt - _vfull(sc_base, jnp.int32)

  def occ_total(cnt):
    return cnt + _vfull(1 - sc_base, jnp.int32)

  def zb_fill(i, _):
    zerobuf[pl.ds(i * L, L)] = jnp.zeros((L,), jnp.float32)
    return 0
  lax.fori_loop(0, ZB // L, zb_fill, 0)

  row0 = wid * nrows
  pltpu.sync_copy(
      params_hbm.at[pl.ds(row0 * 5 * L, nrows * 5 * L)], pbuf)
  stage = pltpu.async_copy(logits_hbm.at[row0], rowbuf, sem_s)

  for r in range(nrows):
    row = row0 + r
    stage.wait()

    p_vec = pbuf[pl.ds((r * 5 + 0) * L, L)]
    a_vec = pbuf[pl.ds((r * 5 + 1) * L, L)]
    m_vec = pbuf[pl.ds((r * 5 + 2) * L, L)]
    k_vec = pbuf[pl.ds((r * 5 + 3) * L, L)].astype(jnp.int32)
    temp_vec = pbuf[pl.ds((r * 5 + 4) * L, L)]

    UA = 10
    def pa2(i, carry):
      mxv, mnv = carry
      base = i * (UA * L)
      vs = [rowbuf[pl.ds(base + u * L, L)] for u in range(UA)]
      mxs, mns = list(vs), list(vs)
      while len(mxs) > 1:
        mxs = [jnp.maximum(mxs[t], mxs[t + 1])
               for t in range(0, len(mxs) - 1, 2)] + (
                   [mxs[-1]] if len(mxs) % 2 else [])
        mns = [jnp.minimum(mns[t], mns[t + 1])
               for t in range(0, len(mns) - 1, 2)] + (
                   [mns[-1]] if len(mns) % 2 else [])
      return jnp.maximum(mxv, mxs[0]), jnp.minimum(mnv, mns[0])
    mxv, mnv = lax.fori_loop(
        0, nv_row // UA, pa2, (_vfull(-jnp.inf), _vfull(jnp.inf)))
    mx = jnp.max(mxv)
    mn = jnp.min(mnv)
    mxr_vec = _vfull(mx)
    inv_temp = _vfull(1.0) / temp_vec
    mx_vec = mxr_vec / temp_vec
    range_vec = jnp.maximum(mxr_vec - _vfull(mn), _vfull(1e-30))
    scale_vec = _vfull(float(NBINS)) / range_vec

    rbase = row * vocab
    zcopies = []
    nfull = vocab // ZB
    for c in range(nfull):
      zcopies.append(pltpu.async_copy(
          zerobuf, out_hbm.at[pl.ds(rbase + c * ZB, ZB)], sem_z))
    tail = vocab - nfull * ZB
    if tail:
      zcopies.append(pltpu.async_copy(
          zerobuf.at[pl.ds(0, tail)],
          out_hbm.at[pl.ds(rbase + nfull * ZB, tail)], sem_z))

    def hz(i, _):
      hist[pl.ds(i * L, L)] = jnp.zeros((L,), jnp.int32)
      return 0
    lax.fori_loop(0, NBINS, hz, 0)

    UB = 10
    ones_i32 = _vfull(1, jnp.int32)
    def pb(i, acc):
      base = i * (UB * L)
      vs = [rowbuf[pl.ds(base + u * L, L)] for u in range(UB)]
      es = [jnp.exp((v - mxr_vec) * inv_temp) for v in vs]
      ixs = [((((mxr_vec - v) * scale_vec).astype(jnp.int32)) << 4) | lane
             for v in vs]
      for u in range(UB):
        plsc.addupdate_scatter(hist, [ixs[u]], ones_i32)
      while len(es) > 1:
        es = [es[t] + es[t + 1] for t in range(0, len(es) - 1, 2)] + (
            [es[-1]] if len(es) % 2 else [])
      return acc + es[0]
    acc = lax.fori_loop(0, nv_row // UB, pb, jnp.zeros((L,), jnp.float32))
    denom = jnp.sum(acc)
    d_vec = _vfull(denom)

    def pf(b, c):
      csum, bstar, found = c
      tot = jnp.sum(hist[pl.ds(b * L, L)])
      csum2 = csum + tot
      hit = jnp.logical_and(csum2 >= TOPK, found == 0)
      bstar = jnp.where(hit, b, bstar)
      return csum2, bstar, found | hit.astype(jnp.int32)
    _, bstar, _ = lax.fori_loop(
        0, NBINS, pf, (jnp.int32(0), jnp.int32(NBINS - 1), jnp.int32(0)))
    tstar_vec = mxr_vec - (
        (_vfull(bstar.astype(jnp.float32)) + _vfull(1.5))
        * range_vec * _vfull(1.0 / NBINS))

    def sf(i, _):
      cva[pl.ds(i * L, L)] = _vfull(_NEG_INF)
      cvb[pl.ds(i * L, L)] = _vfull(_NEG_INF)
      return 0
    lax.fori_loop(0, (CAP + L) // L, sf, 0)

    UC = 10
    def pc(i, cnt):
      base = i * (UC * L)
      vs, sels, sums = [], [], []
      for u in range(UC):
        v = rowbuf[pl.ds(base + u * L, L)]
        sel = v >= tstar_vec
        vs.append(v); sels.append(sel)
        sums.append(jnp.sum(sel.astype(jnp.int32)))
      for u in range(UC):
        off = jnp.minimum(cnt, CAP)
        plsc.store_compressed(cva.at[pl.ds(off, L)], vs[u], mask=sels[u])
        idxv = _vfull(base + u * L, jnp.int32) + lane
        plsc.store_compressed(cia.at[pl.ds(off, L)], idxv, mask=sels[u])
        cnt = cnt + sums[u]
      return cnt
    cnt = lax.fori_loop(0, nv_row // UC, pc, jnp.int32(0))
    nsel = jnp.minimum(cnt, jnp.int32(CAP))
    nvc = (nsel + L - 1) >> 4

    if r + 1 < nrows:
      stage = pltpu.async_copy(logits_hbm.at[row + 1], rowbuf, sem_s)

    def csc(j, _):
      cva[pl.ds(j * L, L)] = cva[pl.ds(j * L, L)] / temp_vec
      return 0
    lax.fori_loop(0, nvc, csc, 0)

    bufs = [(cva, cia), (cvb, cib)]
    for pidx in range(NPASS):
      vsrc, isrc = bufs[pidx % 2]
      vdst, idst = bufs[(pidx + 1) % 2]
      shift = RADIX_BITS * pidx

      def rz(i, _):
        rhist[pl.ds(i * L, L)] = jnp.zeros((L,), jnp.int32)
        return 0
      lax.fori_loop(0, NRAD // L, rz, 0)

      def h1(j, _, vsrc=vsrc, shift=shift):
        d = _sort_digit(vsrc[pl.ds(j * L, L)], shift)
        cnt1, last1 = plsc.scan_count(d)
        plsc.addupdate_scatter(rhist, [d], occ_total(cnt1), mask=last1)
        return 0
      lax.fori_loop(0, nvc, h1, 0)

      def h2(j, c):
        h = rhist[pl.ds(j * L, L)]
        inc = icumsum_i(h)
        offs[pl.ds(j * L, L)] = _vfull(c, jnp.int32) + inc - h
        return c + jnp.sum(h)
      lax.fori_loop(0, NRAD // L, h2, jnp.int32(0))

      def h3(j, _, vsrc=vsrc, isrc=isrc, vdst=vdst, idst=idst, shift=shift):
        v = vsrc[pl.ds(j * L, L)]
        iv = isrc[pl.ds(j * L, L)]
        d = _sort_digit(v, shift)
        cnt3, last3 = plsc.scan_count(d)
        basek = plsc.load_gather(offs, [d])
        pos = basek + occ_rank(cnt3)
        plsc.store_scatter(vdst, [pos], v)
        plsc.store_scatter(idst, [pos], iv)
        plsc.addupdate_scatter(offs, [d], occ_total(cnt3), mask=last3)
        return 0
      lax.fori_loop(0, nvc, h3, 0)

    q0 = _vfull(1.0) / d_vec
    t_vec = jnp.maximum(m_vec * q0, a_vec * q0 * q0)
    zero_v = jnp.zeros((L,), jnp.float32)
    row_off = _vfull(rbase, jnp.int32)

    UF = 4
    def fm(g, c):
      csum, skeepv = c
      js = [g * UF + t for t in range(UF)]
      vs = [cva[pl.ds(j * L, L)] for j in js]
      qs = [jnp.exp(v - mx_vec) / d_vec for v in vs]
      incs = [icumsum_f(q) for q in qs]
      tots = [jnp.sum(q) for q in qs]
      for t in range(UF):
        j = js[t]
        excl = _vfull(csum) + incs[t] - qs[t]
        ranks = _vfull(j * L, jnp.int32) + lane
        keep = (ranks < k_vec) & (
            (ranks == 0) | ((qs[t] >= t_vec) & (excl <= p_vec)))
        qk = jnp.where(keep, qs[t], zero_v)
        skeepv = skeepv + qk
        jj = j >> 3
        col = (j & 7) * L
        qsrc[jj, pl.ds(col, L)] = qk
        gidx[jj, pl.ds(col, L)] = cia[pl.ds(j * L, L)] + row_off
        csum = csum + tots[t]
      return csum, skeepv
    _, skeepv = lax.fori_loop(
        0, TOPK // L // UF, fm,
        (jnp.float32(0.0), jnp.zeros((L,), jnp.float32)))
    skeep_vec = _vfull(jnp.sum(skeepv))

    def fd(g, _):
      for t in range(UF):
        j = g * UF + t
        jj = j >> 3
        col = (j & 7) * L
        qsrc[jj, pl.ds(col, L)] = qsrc[jj, pl.ds(col, L)] / skeep_vec
      return 0
    lax.fori_loop(0, TOPK // L // UF, fd, 0)

    for zc in zcopies:
      zc.wait()
    dcopies = []
    for j in range(TOPK // 128):
      dcopies.append(
          pltpu.async_copy(qsrc.at[j], out_hbm.at[gidx.at[j]], sem_d))
    for dc in dcopies:
      dc.wait()


def kernel(logits, p, k, a, m, temperatures):
  b, v = logits.shape
  nrows = b // NW
  temps = jnp.where(temperatures == 0.0, 1.0, temperatures)
  params = jnp.stack(
      [p, a, m, k.astype(jnp.float32), temps], axis=1)
  params3 = jnp.broadcast_to(params[:, :, None], (b, 5, L))
  params3 = jnp.asarray(params3, jnp.float32).reshape(-1)

  mesh = plsc.VectorSubcoreMesh(
      core_axis_name="c", subcore_axis_name="s",
      num_cores=NC, num_subcores=NS)
  run = pl.kernel(
      functools.partial(_body, nrows, v),
      out_type=jax.ShapeDtypeStruct((b * v,), jnp.float32),
      mesh=mesh,
      scratch_types=[
          pltpu.VMEM((v,), jnp.float32),
          pltpu.VMEM((nrows * 5 * L,), jnp.float32),
          pltpu.VMEM((NBINS * L + 2 * L,), jnp.int32),
          pltpu.VMEM((NRAD,), jnp.int32),
          pltpu.VMEM((NRAD,), jnp.int32),
          pltpu.VMEM((CAP + L,), jnp.float32),
          pltpu.VMEM((CAP + L,), jnp.int32),
          pltpu.VMEM((CAP + L,), jnp.float32),
          pltpu.VMEM((CAP + L,), jnp.int32),
          pltpu.VMEM((TOPK // 128, 128), jnp.float32),
          pltpu.VMEM((TOPK // 128, 128), jnp.int32),
          pltpu.VMEM((ZB,), jnp.float32),
          pltpu.SemaphoreType.DMA,
          pltpu.SemaphoreType.DMA,
          pltpu.SemaphoreType.DMA,
      ],
      compiler_params=pltpu.CompilerParams(needs_layout_passes=False),
  )
  out_flat = run(logits, params3)
  return out_flat.reshape(b, v)


# speedup vs baseline: 62.9987x; 1.0071x over previous
import functools

import jax
import jax.numpy as jnp
from jax import lax
from jax.experimental import pallas as pl
from jax.experimental.pallas import tpu as pltpu
from jax.experimental.pallas import tpu_sc as plsc

L = 16
NC, NS = 2, 16
NW = NC * NS

NBINS = 128
CAP = 2048
TOPK = 1024
RADIX_BITS = 8
NRAD = 1 << RADIX_BITS
NPASS = 4
ZB = 8192

_I32_MIN = -2147483648
_NEG_INF = float("-inf")


def _vfull(x, dtype=jnp.float32):
  return jnp.full((L,), x, dtype=dtype)


def _sort_digit(v, shift):
  b = plsc.bitcast(v, jnp.int32)
  t = jnp.where(b < 0, ~b, b | _I32_MIN)
  key = ~t
  return (key >> shift) & (NRAD - 1)


def _body(nrows, vocab, logits_hbm, params_hbm, out_hbm,
          rowbuf, pbuf, hist, rhist, offs,
          cva, cia, cvb, cib, qsrc, gidx, zerobuf,
          sem_s, sem_z, sem_d):
  nv_row = vocab // L
  lane = lax.iota(jnp.int32, L)
  wid = lax.axis_index("s") * NC + lax.axis_index("c")

  ones_i = _vfull(1, jnp.int32)
  cal_cnt, _ = plsc.scan_count(jnp.zeros((L,), jnp.int32))
  sc_base = jnp.max(jnp.where(lane == 0, cal_cnt, 0))
  cs_probe = plsc.cumsum(ones_i)
  cs_incl = jnp.max(jnp.where(lane == 0, cs_probe, 0))

  def icumsum_i(x):
    return plsc.cumsum(x) + x * _vfull(1 - cs_incl, jnp.int32)

  def icumsum_f(x):
    adj = (jnp.int32(1) - cs_incl).astype(jnp.float32)
    return plsc.cumsum(x) + x * _vfull(adj, jnp.float32)

  def occ_rank(cnt):
    return cnt - _vfull(sc_base, jnp.int32)

  def occ_total(cnt):
    return cnt + _vfull(1 - sc_base, jnp.int32)

  def zb_fill(i, _):
    zerobuf[pl.ds(i * L, L)] = jnp.zeros((L,), jnp.float32)
    return 0
  lax.fori_loop(0, ZB // L, zb_fill, 0)

  row0 = wid * nrows
  pltpu.sync_copy(
      params_hbm.at[pl.ds(row0 * 5 * L, nrows * 5 * L)], pbuf)
  stage = pltpu.async_copy(logits_hbm.at[row0], rowbuf, sem_s)

  for r in range(nrows):
    row = row0 + r
    stage.wait()

    p_vec = pbuf[pl.ds((r * 5 + 0) * L, L)]
    a_vec = pbuf[pl.ds((r * 5 + 1) * L, L)]
    m_vec = pbuf[pl.ds((r * 5 + 2) * L, L)]
    k_vec = pbuf[pl.ds((r * 5 + 3) * L, L)].astype(jnp.int32)
    temp_vec = pbuf[pl.ds((r * 5 + 4) * L, L)]

    UA = 10
    def pa2(i, carry):
      mxv, mnv = carry
      base = i * (UA * L)
      vs = [rowbuf[pl.ds(base + u * L, L)] for u in range(UA)]
      mxs, mns = list(vs), list(vs)
      while len(mxs) > 1:
        mxs = [jnp.maximum(mxs[t], mxs[t + 1])
               for t in range(0, len(mxs) - 1, 2)] + (
                   [mxs[-1]] if len(mxs) % 2 else [])
        mns = [jnp.minimum(mns[t], mns[t + 1])
               for t in range(0, len(mns) - 1, 2)] + (
                   [mns[-1]] if len(mns) % 2 else [])
      return jnp.maximum(mxv, mxs[0]), jnp.minimum(mnv, mns[0])
    mxv, mnv = lax.fori_loop(
        0, nv_row // UA, pa2, (_vfull(-jnp.inf), _vfull(jnp.inf)))
    mx = jnp.max(mxv)
    mn = jnp.min(mnv)
    mxr_vec = _vfull(mx)
    inv_temp = _vfull(1.0) / temp_vec
    mx_vec = mxr_vec / temp_vec
    range_vec = jnp.maximum(mxr_vec - _vfull(mn), _vfull(1e-30))
    scale_vec = _vfull(float(NBINS)) / range_vec

    rbase = row * vocab
    zcopies = []
    nfull = vocab // ZB
    for c in range(nfull):
      zcopies.append(pltpu.async_copy(
          zerobuf, out_hbm.at[pl.ds(rbase + c * ZB, ZB)], sem_z))
    tail = vocab - nfull * ZB
    if tail:
      zcopies.append(pltpu.async_copy(
          zerobuf.at[pl.ds(0, tail)],
          out_hbm.at[pl.ds(rbase + nfull * ZB, tail)], sem_z))

    def hz(i, _):
      hist[pl.ds(i * L, L)] = jnp.zeros((L,), jnp.int32)
      return 0
    lax.fori_loop(0, NBINS, hz, 0)

    UB = 10
    ones_i32 = _vfull(1, jnp.int32)
    nscale_vec = -scale_vec
    def pb(i, acc):
      base = i * (UB * L)
      vs = [rowbuf[pl.ds(base + u * L, L)] for u in range(UB)]
      dsub = [v - mxr_vec for v in vs]
      es = [jnp.exp(d * inv_temp) for d in dsub]
      ixs = [(((d * nscale_vec).astype(jnp.int32)) << 4) | lane
             for d in dsub]
      for u in range(UB):
        plsc.addupdate_scatter(hist, [ixs[u]], ones_i32)
      while len(es) > 1:
        es = [es[t] + es[t + 1] for t in range(0, len(es) - 1, 2)] + (
            [es[-1]] if len(es) % 2 else [])
      return acc + es[0]
    acc = lax.fori_loop(0, nv_row // UB, pb, jnp.zeros((L,), jnp.float32))
    denom = jnp.sum(acc)
    d_vec = _vfull(denom)

    UP = 4
    def pf(g, c):
      csum, bstar, found = c
      tots = [jnp.sum(hist[pl.ds((g * UP + t) * L, L)]) for t in range(UP)]
      for t in range(UP):
        csum = csum + tots[t]
        hit = jnp.logical_and(csum >= TOPK, found == 0)
        bstar = jnp.where(hit, g * UP + t, bstar)
        found = found | hit.astype(jnp.int32)
      return csum, bstar, found
    _, bstar, _ = lax.fori_loop(
        0, NBINS // UP, pf,
        (jnp.int32(0), jnp.int32(NBINS - 1), jnp.int32(0)))
    tstar_vec = mxr_vec - (
        (_vfull(bstar.astype(jnp.float32)) + _vfull(1.5))
        * range_vec * _vfull(1.0 / NBINS))

    def sf(i, _):
      cva[pl.ds(i * L, L)] = _vfull(_NEG_INF)
      cvb[pl.ds(i * L, L)] = _vfull(_NEG_INF)
      return 0
    lax.fori_loop(0, (CAP + L) // L, sf, 0)

    UC = 10
    def pc(i, cnt):
      base = i * (UC * L)
      vs, sels, sums = [], [], []
      for u in range(UC):
        v = rowbuf[pl.ds(base + u * L, L)]
        sel = v >= tstar_vec
        vs.append(v); sels.append(sel)
        sums.append(jnp.sum(sel.astype(jnp.int32)))
      for u in range(UC):
        off = jnp.minimum(cnt, CAP)
        plsc.store_compressed(cva.at[pl.ds(off, L)], vs[u], mask=sels[u])
        idxv = _vfull(base + u * L, jnp.int32) + lane
        plsc.store_compressed(cia.at[pl.ds(off, L)], idxv, mask=sels[u])
        cnt = cnt + sums[u]
      return cnt
    cnt = lax.fori_loop(0, nv_row // UC, pc, jnp.int32(0))
    nsel = jnp.minimum(cnt, jnp.int32(CAP))
    nvc = (nsel + L - 1) >> 4

    if r + 1 < nrows:
      stage = pltpu.async_copy(logits_hbm.at[row + 1], rowbuf, sem_s)

    def csc(j, _):
      cva[pl.ds(j * L, L)] = cva[pl.ds(j * L, L)] / temp_vec
      return 0
    lax.fori_loop(0, nvc, csc, 0)

    bufs = [(cva, cia), (cvb, cib)]
    for pidx in range(NPASS):
      vsrc, isrc = bufs[pidx % 2]
      vdst, idst = bufs[(pidx + 1) % 2]
      shift = RADIX_BITS * pidx

      def rz(i, _):
        rhist[pl.ds(i * L, L)] = jnp.zeros((L,), jnp.int32)
        return 0
      lax.fori_loop(0, NRAD // L, rz, 0)

      def h1(j, _, vsrc=vsrc, shift=shift):
        d = _sort_digit(vsrc[pl.ds(j * L, L)], shift)
        cnt1, last1 = plsc.scan_count(d)
        plsc.addupdate_scatter(rhist, [d], occ_total(cnt1), mask=last1)
        return 0
      lax.fori_loop(0, nvc, h1, 0)

      def h2(j, c):
        h = rhist[pl.ds(j * L, L)]
        inc = icumsum_i(h)
        offs[pl.ds(j * L, L)] = _vfull(c, jnp.int32) + inc - h
        return c + jnp.sum(h)
      lax.fori_loop(0, NRAD // L, h2, jnp.int32(0))

      def h3(j, _, vsrc=vsrc, isrc=isrc, vdst=vdst, idst=idst, shift=shift):
        v = vsrc[pl.ds(j * L, L)]
        iv = isrc[pl.ds(j * L, L)]
        d = _sort_digit(v, shift)
        cnt3, last3 = plsc.scan_count(d)
        basek = plsc.load_gather(offs, [d])
        pos = basek + occ_rank(cnt3)
        plsc.store_scatter(vdst, [pos], v)
        plsc.store_scatter(idst, [pos], iv)
        plsc.addupdate_scatter(offs, [d], occ_total(cnt3), mask=last3)
        return 0
      lax.fori_loop(0, nvc, h3, 0)

    q0 = _vfull(1.0) / d_vec
    t_vec = jnp.maximum(m_vec * q0, a_vec * q0 * q0)
    zero_v = jnp.zeros((L,), jnp.float32)
    row_off = _vfull(rbase, jnp.int32)

    UF = 4
    def fm(g, c):
      csum, skeepv = c
      js = [g * UF + t for t in range(UF)]
      vs = [cva[pl.ds(j * L, L)] for j in js]
      qs = [jnp.exp(v - mx_vec) / d_vec for v in vs]
      incs = [icumsum_f(q) for q in qs]
      tots = [jnp.sum(q) for q in qs]
      for t in range(UF):
        j = js[t]
        excl = _vfull(csum) + incs[t] - qs[t]
        ranks = _vfull(j * L, jnp.int32) + lane
        keep = (ranks < k_vec) & (
            (ranks == 0) | ((qs[t] >= t_vec) & (excl <= p_vec)))
        qk = jnp.where(keep, qs[t], zero_v)
        skeepv = skeepv + qk
        jj = j >> 3
        col = (j & 7) * L
        qsrc[jj, pl.ds(col, L)] = qk
        gidx[jj, pl.ds(col, L)] = cia[pl.ds(j * L, L)] + row_off
        csum = csum + tots[t]
      return csum, skeepv
    _, skeepv = lax.fori_loop(
        0, TOPK // L // UF, fm,
        (jnp.float32(0.0), jnp.zeros((L,), jnp.float32)))
    skeep_vec = _vfull(jnp.sum(skeepv))

    def fd(g, _):
      for t in range(UF):
        j = g * UF + t
        jj = j >> 3
        col = (j & 7) * L
        qsrc[jj, pl.ds(col, L)] = qsrc[jj, pl.ds(col, L)] / skeep_vec
      return 0
    lax.fori_loop(0, TOPK // L // UF, fd, 0)

    for zc in zcopies:
      zc.wait()
    dcopies = []
    for j in range(TOPK // 128):
      dcopies.append(
          pltpu.async_copy(qsrc.at[j], out_hbm.at[gidx.at[j]], sem_d))
    for dc in dcopies:
      dc.wait()


def kernel(logits, p, k, a, m, temperatures):
  b, v = logits.shape
  nrows = b // NW
  temps = jnp.where(temperatures == 0.0, 1.0, temperatures)
  params = jnp.stack(
      [p, a, m, k.astype(jnp.float32), temps], axis=1)
  params3 = jnp.broadcast_to(params[:, :, None], (b, 5, L))
  params3 = jnp.asarray(params3, jnp.float32).reshape(-1)

  mesh = plsc.VectorSubcoreMesh(
      core_axis_name="c", subcore_axis_name="s",
      num_cores=NC, num_subcores=NS)
  run = pl.kernel(
      functools.partial(_body, nrows, v),
      out_type=jax.ShapeDtypeStruct((b * v,), jnp.float32),
      mesh=mesh,
      scratch_types=[
          pltpu.VMEM((v,), jnp.float32),
          pltpu.VMEM((nrows * 5 * L,), jnp.float32),
          pltpu.VMEM((NBINS * L + 2 * L,), jnp.int32),
          pltpu.VMEM((NRAD,), jnp.int32),
          pltpu.VMEM((NRAD,), jnp.int32),
          pltpu.VMEM((CAP + L,), jnp.float32),
          pltpu.VMEM((CAP + L,), jnp.int32),
          pltpu.VMEM((CAP + L,), jnp.float32),
          pltpu.VMEM((CAP + L,), jnp.int32),
          pltpu.VMEM((TOPK // 128, 128), jnp.float32),
          pltpu.VMEM((TOPK // 128, 128), jnp.int32),
          pltpu.VMEM((ZB,), jnp.float32),
          pltpu.SemaphoreType.DMA,
          pltpu.SemaphoreType.DMA,
          pltpu.SemaphoreType.DMA,
      ],
      compiler_params=pltpu.CompilerParams(needs_layout_passes=False),
  )
  out_flat = run(logits, params3)
  return out_flat.reshape(b, v)
